# baseline (speedup 1.0000x reference)
import jax
import jax.numpy as jnp
from jax.experimental import pallas as pl
from jax.experimental.pallas import tpu as pltpu

_BM = 200
_NBUF = 4


def _outer(alpha_ref, m_hbm, x_hbm, h_hbm, ah_hbm):
    n, k = m_hbm.shape
    d = x_hbm.shape[1]

    def _inner(m_ref, x_ref, h_ref, ah_ref):
        h = jax.lax.dot_general(
            m_ref[...],
            x_ref[...],
            dimension_numbers=(((1,), (0,)), ((), ())),
            preferred_element_type=jnp.float32,
        )
        h_ref[...] = h
        ah_ref[...] = alpha_ref[0] * h

    pipeline = pltpu.emit_pipeline(
        _inner,
        grid=(n // _BM,),
        in_specs=[
            pl.BlockSpec(
                (_BM, k), lambda i: (i, 0),
                pipeline_mode=pl.Buffered(buffer_count=_NBUF),
            ),
            pl.BlockSpec((k, d), lambda i: (0, 0)),
        ],
        out_specs=[
            pl.BlockSpec((_BM, d), lambda i: (i, 0)),
            pl.BlockSpec((_BM, d), lambda i: (i, 0)),
        ],
    )
    pipeline(m_hbm, x_hbm, h_hbm, ah_hbm)


def kernel(inp, M, alpha):
    n, k = M.shape
    d = inp.shape[1]
    out = pl.pallas_call(
        _outer,
        in_specs=[
            pl.BlockSpec(memory_space=pltpu.SMEM),
            pl.BlockSpec(memory_space=pl.ANY),
            pl.BlockSpec(memory_space=pl.ANY),
        ],
        out_specs=[
            pl.BlockSpec(memory_space=pl.ANY),
            pl.BlockSpec(memory_space=pl.ANY),
        ],
        out_shape=[
            jax.ShapeDtypeStruct((n, d), jnp.float32),
            jax.ShapeDtypeStruct((n, d), jnp.float32),
        ],
    )(alpha, M, inp)
    return (out[0], out[1])
